# baseline (speedup 1.0000x reference)
import functools

import jax
import jax.numpy as jnp
from jax import lax
from jax.experimental import pallas as pl
from jax.experimental.pallas import tpu as pltpu
from jax.experimental.pallas import tpu_sc as plsc

N_NODES = 10000
N_EDGES = 320000
DIM = 128
NUM_REL = 8
K = NUM_REL * N_NODES

NC = 2
NS = 16
NW = NC * NS
EDGES_PER_W = N_EDGES // NW
CHUNK = 80
NCHUNK = EDGES_PER_W // CHUNK

KPAD = 81920
DPAD = 10240


def _mm_body(x_ref, w_ref, c_ref, g_ref, b_ref, y_ref, ab_ref, t1_ref, t2_ref):
    i = pl.program_id(0)

    @pl.when(i == 0)
    def _():
        t1_ref[...] = jnp.zeros_like(t1_ref)
        t2_ref[...] = jnp.zeros_like(t2_ref)

    yb = jnp.maximum(
        jnp.dot(x_ref[...], w_ref[...], preferred_element_type=jnp.float32), 0.0)
    y_ref[...] = yb

    cm = jnp.sum(c_ref[...], axis=0)
    r1 = jnp.dot(cm, yb, preferred_element_type=jnp.float32)
    r2 = jnp.dot(cm, yb * yb, preferred_element_type=jnp.float32)
    for r in range(NUM_REL):
        t1_ref[...] += r1[r:r + 1, r * DIM:(r + 1) * DIM]
        t2_ref[...] += r2[r:r + 1, r * DIM:(r + 1) * DIM]

    @pl.when(i == pl.num_programs(0) - 1)
    def _():
        mean = t1_ref[...] / N_EDGES
        var = t2_ref[...] / N_EDGES - mean * mean
        a = g_ref[...] * lax.rsqrt(var + 1e-5)
        b = b_ref[...] - mean * a
        ab_ref[0:1, :] = a
        ab_ref[1:2, :] = b


def _relu_xw_stats(x, wc, c5, gamma, beta):
    return pl.pallas_call(
        _mm_body,
        grid=(10,),
        in_specs=[
            pl.BlockSpec((1000, DIM), lambda i: (i, 0)),
            pl.BlockSpec((DIM, NUM_REL * DIM), lambda i: (0, 0)),
            pl.BlockSpec((NW, NUM_REL, 1000), lambda i: (0, i, 0)),
            pl.BlockSpec((1, DIM), lambda i: (0, 0)),
            pl.BlockSpec((1, DIM), lambda i: (0, 0)),
        ],
        out_specs=[
            pl.BlockSpec((1000, NUM_REL * DIM), lambda i: (i, 0)),
            pl.BlockSpec((2, DIM), lambda i: (0, 0)),
        ],
        out_shape=[
            jax.ShapeDtypeStruct((N_NODES, NUM_REL * DIM), jnp.float32),
            jax.ShapeDtypeStruct((2, DIM), jnp.float32),
        ],
        scratch_shapes=[
            pltpu.VMEM((1, DIM), jnp.float32),
            pltpu.VMEM((1, DIM), jnp.float32),
        ],
    )(x, wc, c5, gamma, beta)


def _sc_hist_kernel(srcrel_hbm, dst_hbm, c_out, deg_out,
                    hist_c, hist_deg, srcrel_v, dst_v):
    c = lax.axis_index("c")
    s = lax.axis_index("s")
    wid = c * NS + s

    ones = jnp.full((16,), 1.0, jnp.float32)

    def _zhc(i, _):
        for g in range(8):
            hist_c[pl.ds(i * 128 + g * 16, 16)] = jnp.zeros((16,), jnp.float32)
        return _
    lax.fori_loop(0, KPAD // 128, _zhc, None)

    def _zhd(i, _):
        for g in range(8):
            hist_deg[pl.ds(i * 128 + g * 16, 16)] = jnp.zeros((16,), jnp.float32)
        return _
    lax.fori_loop(0, DPAD // 128, _zhd, None)

    pltpu.sync_copy(srcrel_hbm.at[pl.ds(wid * EDGES_PER_W, EDGES_PER_W)], srcrel_v)
    pltpu.sync_copy(dst_hbm.at[pl.ds(wid * EDGES_PER_W, EDGES_PER_W)], dst_v)

    def _edge(g, _):
        sr = srcrel_v[pl.ds(g * 16, 16)]
        plsc.addupdate_scatter(hist_c, [sr], ones)
        dv = dst_v[pl.ds(g * 16, 16)]
        plsc.addupdate_scatter(hist_deg, [dv], ones)
        return _
    lax.fori_loop(0, EDGES_PER_W // 16, _edge, None)

    pltpu.sync_copy(hist_c, c_out.at[pl.ds(wid * KPAD, KPAD)])
    pltpu.sync_copy(hist_deg, deg_out.at[pl.ds(wid * DPAD, DPAD)])


def _sc_hist_stage(srcrel, dst):
    mesh = plsc.VectorSubcoreMesh(core_axis_name="c", subcore_axis_name="s")
    kern = functools.partial(
        pl.kernel,
        out_type=[
            jax.ShapeDtypeStruct((NW * KPAD,), jnp.float32),
            jax.ShapeDtypeStruct((NW * DPAD,), jnp.float32),
        ],
        mesh=mesh,
        compiler_params=pltpu.CompilerParams(needs_layout_passes=False),
        scratch_types=[
            pltpu.VMEM((KPAD,), jnp.float32),
            pltpu.VMEM((DPAD,), jnp.float32),
            pltpu.VMEM((EDGES_PER_W,), jnp.int32),
            pltpu.VMEM((EDGES_PER_W,), jnp.int32),
        ],
    )(_sc_hist_kernel)
    return kern(srcrel, dst)


CH = 80
NCH = EDGES_PER_W // CH


def _sc_acc_kernel(srcrel_hbm, dst_hbm, y_hbm, acc_out,
                   acc_sh, r0, r1, r2, i0, i1, i2, dst_v,
                   g0, g1, g2, s0, s1, s2, p0, p1, p2):
    c = lax.axis_index("c")
    s = lax.axis_index("s")
    wid = c * NS + s
    ebase = wid * EDGES_PER_W

    rbuf = (r0, r1, r2)
    ibuf = (i0, i1, i2)
    gsem = (g0, g1, g2)
    ssem = (s0, s1, s2)
    isem = (p0, p1, p2)

    def _zrow(i, _):
        for g in range(8):
            r0[i, pl.ds(g * 16, 16)] = jnp.zeros((16,), jnp.float32)
        return _
    lax.fori_loop(0, CH, _zrow, None)

    def _zacc(i, _):
        blk = s + NS * i
        @pl.when(blk < N_NODES // 80)
        def _():
            pltpu.sync_copy(r0.at[pl.ds(0, 80)], acc_sh.at[pl.ds(blk * 80, 80)])
        return _
    lax.fori_loop(0, (N_NODES // 80 + NS - 1) // NS, _zacc, None)

    pltpu.sync_copy(dst_hbm.at[wid], dst_v)

    plsc.subcore_barrier()

    def _idx_load(i, k):
        pltpu.async_copy(srcrel_hbm.at[pl.ds(ebase + i * CH, CH)], ibuf[k], isem[k])

    def _idx_wait(i, k):
        pltpu.make_async_copy(
            srcrel_hbm.at[pl.ds(ebase + i * CH, CH)], ibuf[k], isem[k]).wait()

    _idx_load(0, 0)
    _idx_load(1, 1)
    _idx_wait(0, 0)
    pltpu.async_copy(y_hbm.at[ibuf[0]], rbuf[0], gsem[0])

    def _emit(i, k):
        k1 = (k + 1) % 3
        k2 = (k + 2) % 3

        @pl.when(i >= 2)
        def _():
            pltpu.make_async_copy(rbuf[k1], acc_sh.at[dst_v.at[i - 2]],
                                  ssem[k1]).wait()
        _idx_wait(i + 1, k1)
        pltpu.async_copy(y_hbm.at[ibuf[k1]], rbuf[k1], gsem[k1])
        _idx_load(i + 2, k2)
        pltpu.make_async_copy(y_hbm.at[ibuf[k]], rbuf[k], gsem[k]).wait()
        pltpu.async_copy(rbuf[k], acc_sh.at[dst_v.at[i]], ssem[k], add=True)

    def _step(j, _):
        _emit(3 * j, 0)
        _emit(3 * j + 1, 1)
        _emit(3 * j + 2, 2)
        return _
    lax.fori_loop(0, (NCH - 2) // 3, _step, None)

    pltpu.make_async_copy(rbuf[1], acc_sh.at[dst_v.at[121]], ssem[1]).wait()
    _idx_wait(124, 1)
    pltpu.async_copy(y_hbm.at[ibuf[1]], rbuf[1], gsem[1])
    pltpu.make_async_copy(y_hbm.at[ibuf[0]], rbuf[0], gsem[0]).wait()
    pltpu.async_copy(rbuf[0], acc_sh.at[dst_v.at[123]], ssem[0], add=True)

    pltpu.make_async_copy(rbuf[2], acc_sh.at[dst_v.at[122]], ssem[2]).wait()
    pltpu.make_async_copy(y_hbm.at[ibuf[1]], rbuf[1], gsem[1]).wait()
    pltpu.async_copy(rbuf[1], acc_sh.at[dst_v.at[124]], ssem[1], add=True)

    pltpu.make_async_copy(rbuf[0], acc_sh.at[dst_v.at[123]], ssem[0]).wait()
    pltpu.make_async_copy(rbuf[1], acc_sh.at[dst_v.at[124]], ssem[1]).wait()

    plsc.subcore_barrier()

    @pl.when(s < 15)
    def _():
        pltpu.sync_copy(acc_sh.at[pl.ds(s * 640, 640)],
                        acc_out.at[c, pl.ds(s * 640, 640)])

    @pl.when(s == 15)
    def _():
        pltpu.sync_copy(acc_sh.at[pl.ds(9600, 400)],
                        acc_out.at[c, pl.ds(9600, 400)])


def _sc_acc_stage(srcrel3, dst3, y):
    mesh = plsc.VectorSubcoreMesh(core_axis_name="c", subcore_axis_name="s")
    kern = functools.partial(
        pl.kernel,
        out_type=jax.ShapeDtypeStruct((NC, N_NODES, DIM), jnp.float32),
        mesh=mesh,
        compiler_params=pltpu.CompilerParams(needs_layout_passes=False),
        scratch_types=[
            pltpu.VMEM_SHARED((N_NODES, DIM), jnp.float32),
            pltpu.VMEM((CH, DIM), jnp.float32),
            pltpu.VMEM((CH, DIM), jnp.float32),
            pltpu.VMEM((CH, DIM), jnp.float32),
            pltpu.VMEM((CH,), jnp.int32),
            pltpu.VMEM((CH,), jnp.int32),
            pltpu.VMEM((CH,), jnp.int32),
            pltpu.VMEM((NCH, CH), jnp.int32),
        ] + [pltpu.SemaphoreType.DMA] * 9,
    )(_sc_acc_kernel)
    return kern(srcrel3, dst3, y)


def _c2_body(acc_ref, deg_ref, ab_ref, out_ref):
    sums = acc_ref[0] + acc_ref[1]
    d = deg_ref[...]
    a = ab_ref[0:1, :]
    b = ab_ref[1:2, :]
    safe = jnp.where(d > 0, d, 1.0)
    out_ref[...] = jnp.where(d > 0, (sums / safe) * a + b, 0.0)


def _c2_stage(acc, deg2, ab):
    return pl.pallas_call(
        _c2_body,
        grid=(10,),
        in_specs=[
            pl.BlockSpec((2, 1000, DIM), lambda i: (0, i, 0)),
            pl.BlockSpec((1000, 1), lambda i: (i, 0)),
            pl.BlockSpec((2, DIM), lambda i: (0, 0)),
        ],
        out_specs=pl.BlockSpec((1000, DIM), lambda i: (i, 0)),
        out_shape=jax.ShapeDtypeStruct((N_NODES, DIM), jnp.float32),
    )(acc, deg2, ab)


def kernel(x, edge_index, edge_type, weight, bn_gamma, bn_beta):
    src = edge_index[0].astype(jnp.int32)
    dst = edge_index[1].astype(jnp.int32)
    rel = edge_type.astype(jnp.int32)
    srcrel = src * jnp.int32(NUM_REL) + rel
    chist = rel * jnp.int32(DPAD) + src

    c_t, deg_t = _sc_hist_stage(chist, dst)
    c5 = (c_t.reshape(NW, NUM_REL, DPAD)[:, :, :N_NODES]
          .reshape(NW, NUM_REL, 10, 1000).transpose(0, 2, 1, 3)
          .reshape(NW, 10 * NUM_REL, 1000))
    wc = weight.transpose(1, 0, 2).reshape(DIM, NUM_REL * DIM)
    y, ab = _relu_xw_stats(x, wc, c5, bn_gamma.reshape(1, DIM),
                           bn_beta.reshape(1, DIM))
    acc = _sc_acc_stage(srcrel, dst.reshape(NW, NCH, CH), y.reshape(K, DIM))

    deg = deg_t.reshape(NW, DPAD).sum(axis=0)[:N_NODES].reshape(N_NODES, 1)
    return _c2_stage(acc, deg, ab)


# speedup vs baseline: 1.1445x; 1.1445x over previous
import functools

import jax
import jax.numpy as jnp
from jax import lax
from jax.experimental import pallas as pl
from jax.experimental.pallas import tpu as pltpu
from jax.experimental.pallas import tpu_sc as plsc

N_NODES = 10000
N_EDGES = 320000
DIM = 128
NUM_REL = 8
K = NUM_REL * N_NODES

NC = 2
NS = 16
NW = NC * NS
EDGES_PER_W = N_EDGES // NW
CHUNK = 80
NCHUNK = EDGES_PER_W // CHUNK

KPAD = 81920
DPAD = 10240


def _mm_body(x_ref, w_ref, c_ref, g_ref, b_ref, y_ref, ab_ref, t1_ref, t2_ref):
    i = pl.program_id(0)

    @pl.when(i == 0)
    def _():
        t1_ref[...] = jnp.zeros_like(t1_ref)
        t2_ref[...] = jnp.zeros_like(t2_ref)

    yb = jnp.maximum(
        jnp.dot(x_ref[...], w_ref[...], preferred_element_type=jnp.float32), 0.0)
    y_ref[...] = yb

    cm = jnp.sum(c_ref[...], axis=0)
    r1 = jnp.dot(cm, yb, preferred_element_type=jnp.float32)
    r2 = jnp.dot(cm, yb * yb, preferred_element_type=jnp.float32)
    for r in range(NUM_REL):
        t1_ref[...] += r1[r:r + 1, r * DIM:(r + 1) * DIM]
        t2_ref[...] += r2[r:r + 1, r * DIM:(r + 1) * DIM]

    @pl.when(i == pl.num_programs(0) - 1)
    def _():
        mean = t1_ref[...] / N_EDGES
        var = t2_ref[...] / N_EDGES - mean * mean
        a = g_ref[...] * lax.rsqrt(var + 1e-5)
        b = b_ref[...] - mean * a
        ab_ref[0:1, :] = a
        ab_ref[1:2, :] = b


def _relu_xw_stats(x, wc, c5, gamma, beta):
    return pl.pallas_call(
        _mm_body,
        grid=(10,),
        in_specs=[
            pl.BlockSpec((1024, DIM), lambda i: (i, 0)),
            pl.BlockSpec((DIM, NUM_REL * DIM), lambda i: (0, 0)),
            pl.BlockSpec((NW, NUM_REL, 1024), lambda i: (0, 0, i)),
            pl.BlockSpec((1, DIM), lambda i: (0, 0)),
            pl.BlockSpec((1, DIM), lambda i: (0, 0)),
        ],
        out_specs=[
            pl.BlockSpec((1024, NUM_REL * DIM), lambda i: (i, 0)),
            pl.BlockSpec((2, DIM), lambda i: (0, 0)),
        ],
        out_shape=[
            jax.ShapeDtypeStruct((DPAD, NUM_REL * DIM), jnp.float32),
            jax.ShapeDtypeStruct((2, DIM), jnp.float32),
        ],
        scratch_shapes=[
            pltpu.VMEM((1, DIM), jnp.float32),
            pltpu.VMEM((1, DIM), jnp.float32),
        ],
    )(x, wc, c5, gamma, beta)


def _sc_hist_kernel(srcrel_hbm, dst_hbm, c_out, deg_out,
                    hist_c, hist_deg, srcrel_v, dst_v):
    c = lax.axis_index("c")
    s = lax.axis_index("s")
    wid = c * NS + s

    ones = jnp.full((16,), 1.0, jnp.float32)

    def _zhc(i, _):
        for g in range(8):
            hist_c[pl.ds(i * 128 + g * 16, 16)] = jnp.zeros((16,), jnp.float32)
        return _
    lax.fori_loop(0, KPAD // 128, _zhc, None)

    def _zhd(i, _):
        for g in range(8):
            hist_deg[pl.ds(i * 128 + g * 16, 16)] = jnp.zeros((16,), jnp.float32)
        return _
    lax.fori_loop(0, DPAD // 128, _zhd, None)

    pltpu.sync_copy(srcrel_hbm.at[pl.ds(wid * EDGES_PER_W, EDGES_PER_W)], srcrel_v)
    pltpu.sync_copy(dst_hbm.at[pl.ds(wid * EDGES_PER_W, EDGES_PER_W)], dst_v)

    def _edge(g, _):
        sr = srcrel_v[pl.ds(g * 16, 16)]
        plsc.addupdate_scatter(hist_c, [sr], ones)
        dv = dst_v[pl.ds(g * 16, 16)]
        plsc.addupdate_scatter(hist_deg, [dv], ones)
        return _
    lax.fori_loop(0, EDGES_PER_W // 16, _edge, None)

    pltpu.sync_copy(hist_c, c_out.at[pl.ds(wid * KPAD, KPAD)])
    pltpu.sync_copy(hist_deg, deg_out.at[pl.ds(wid * DPAD, DPAD)])


def _sc_hist_stage(srcrel, dst):
    mesh = plsc.VectorSubcoreMesh(core_axis_name="c", subcore_axis_name="s")
    kern = functools.partial(
        pl.kernel,
        out_type=[
            jax.ShapeDtypeStruct((NW * KPAD,), jnp.float32),
            jax.ShapeDtypeStruct((NW * DPAD,), jnp.float32),
        ],
        mesh=mesh,
        compiler_params=pltpu.CompilerParams(needs_layout_passes=False),
        scratch_types=[
            pltpu.VMEM((KPAD,), jnp.float32),
            pltpu.VMEM((DPAD,), jnp.float32),
            pltpu.VMEM((EDGES_PER_W,), jnp.int32),
            pltpu.VMEM((EDGES_PER_W,), jnp.int32),
        ],
    )(_sc_hist_kernel)
    return kern(srcrel, dst)


CH = 80
NCH = EDGES_PER_W // CH


def _sc_acc_kernel(srcrel_hbm, dst_hbm, y_hbm, acc_out,
                   acc_sh, r0, r1, r2, i0, i1, i2, dst_v,
                   g0, g1, g2, s0, s1, s2, p0, p1, p2):
    c = lax.axis_index("c")
    s = lax.axis_index("s")
    wid = c * NS + s
    ebase = wid * EDGES_PER_W

    rbuf = (r0, r1, r2)
    ibuf = (i0, i1, i2)
    gsem = (g0, g1, g2)
    ssem = (s0, s1, s2)
    isem = (p0, p1, p2)

    def _zrow(i, _):
        for g in range(8):
            r0[i, pl.ds(g * 16, 16)] = jnp.zeros((16,), jnp.float32)
        return _
    lax.fori_loop(0, CH, _zrow, None)

    def _zacc(i, _):
        blk = s + NS * i
        @pl.when(blk < N_NODES // 80)
        def _():
            pltpu.sync_copy(r0.at[pl.ds(0, 80)], acc_sh.at[pl.ds(blk * 80, 80)])
        return _
    lax.fori_loop(0, (N_NODES // 80 + NS - 1) // NS, _zacc, None)

    pltpu.sync_copy(dst_hbm.at[wid], dst_v)

    plsc.subcore_barrier()

    def _idx_load(i, k):
        pltpu.async_copy(srcrel_hbm.at[pl.ds(ebase + i * CH, CH)], ibuf[k], isem[k])

    def _idx_wait(i, k):
        pltpu.make_async_copy(
            srcrel_hbm.at[pl.ds(ebase + i * CH, CH)], ibuf[k], isem[k]).wait()

    _idx_load(0, 0)
    _idx_load(1, 1)
    _idx_wait(0, 0)
    pltpu.async_copy(y_hbm.at[ibuf[0]], rbuf[0], gsem[0])

    def _emit(i, k):
        k1 = (k + 1) % 3
        k2 = (k + 2) % 3

        @pl.when(i >= 2)
        def _():
            pltpu.make_async_copy(rbuf[k1], acc_sh.at[dst_v.at[i - 2]],
                                  ssem[k1]).wait()
        _idx_wait(i + 1, k1)
        pltpu.async_copy(y_hbm.at[ibuf[k1]], rbuf[k1], gsem[k1])
        _idx_load(i + 2, k2)
        pltpu.make_async_copy(y_hbm.at[ibuf[k]], rbuf[k], gsem[k]).wait()
        pltpu.async_copy(rbuf[k], acc_sh.at[dst_v.at[i]], ssem[k], add=True)

    def _step(j, _):
        _emit(3 * j, 0)
        _emit(3 * j + 1, 1)
        _emit(3 * j + 2, 2)
        return _
    lax.fori_loop(0, (NCH - 2) // 3, _step, None)

    pltpu.make_async_copy(rbuf[1], acc_sh.at[dst_v.at[121]], ssem[1]).wait()
    _idx_wait(124, 1)
    pltpu.async_copy(y_hbm.at[ibuf[1]], rbuf[1], gsem[1])
    pltpu.make_async_copy(y_hbm.at[ibuf[0]], rbuf[0], gsem[0]).wait()
    pltpu.async_copy(rbuf[0], acc_sh.at[dst_v.at[123]], ssem[0], add=True)

    pltpu.make_async_copy(rbuf[2], acc_sh.at[dst_v.at[122]], ssem[2]).wait()
    pltpu.make_async_copy(y_hbm.at[ibuf[1]], rbuf[1], gsem[1]).wait()
    pltpu.async_copy(rbuf[1], acc_sh.at[dst_v.at[124]], ssem[1], add=True)

    pltpu.make_async_copy(rbuf[0], acc_sh.at[dst_v.at[123]], ssem[0]).wait()
    pltpu.make_async_copy(rbuf[1], acc_sh.at[dst_v.at[124]], ssem[1]).wait()

    plsc.subcore_barrier()

    @pl.when(s < 15)
    def _():
        pltpu.sync_copy(acc_sh.at[pl.ds(s * 640, 640)],
                        acc_out.at[c, pl.ds(s * 640, 640)])

    @pl.when(s == 15)
    def _():
        pltpu.sync_copy(acc_sh.at[pl.ds(9600, 400)],
                        acc_out.at[c, pl.ds(9600, 400)])


def _sc_acc_stage(srcrel3, dst3, y):
    mesh = plsc.VectorSubcoreMesh(core_axis_name="c", subcore_axis_name="s")
    kern = functools.partial(
        pl.kernel,
        out_type=jax.ShapeDtypeStruct((NC, N_NODES, DIM), jnp.float32),
        mesh=mesh,
        compiler_params=pltpu.CompilerParams(needs_layout_passes=False),
        scratch_types=[
            pltpu.VMEM_SHARED((N_NODES, DIM), jnp.float32),
            pltpu.VMEM((CH, DIM), jnp.float32),
            pltpu.VMEM((CH, DIM), jnp.float32),
            pltpu.VMEM((CH, DIM), jnp.float32),
            pltpu.VMEM((CH,), jnp.int32),
            pltpu.VMEM((CH,), jnp.int32),
            pltpu.VMEM((CH,), jnp.int32),
            pltpu.VMEM((NCH, CH), jnp.int32),
        ] + [pltpu.SemaphoreType.DMA] * 9,
    )(_sc_acc_kernel)
    return kern(srcrel3, dst3, y)


def _c2_body(acc_ref, deg_ref, ab_ref, out_ref):
    sums = acc_ref[0] + acc_ref[1]
    d = deg_ref[...]
    a = ab_ref[0:1, :]
    b = ab_ref[1:2, :]
    safe = jnp.where(d > 0, d, 1.0)
    out_ref[...] = jnp.where(d > 0, (sums / safe) * a + b, 0.0)


def _c2_stage(acc, deg2, ab):
    return pl.pallas_call(
        _c2_body,
        grid=(10,),
        in_specs=[
            pl.BlockSpec((2, 1000, DIM), lambda i: (0, i, 0)),
            pl.BlockSpec((1000, 1), lambda i: (i, 0)),
            pl.BlockSpec((2, DIM), lambda i: (0, 0)),
        ],
        out_specs=pl.BlockSpec((1000, DIM), lambda i: (i, 0)),
        out_shape=jax.ShapeDtypeStruct((N_NODES, DIM), jnp.float32),
    )(acc, deg2, ab)


def kernel(x, edge_index, edge_type, weight, bn_gamma, bn_beta):
    src = edge_index[0].astype(jnp.int32)
    dst = edge_index[1].astype(jnp.int32)
    rel = edge_type.astype(jnp.int32)
    srcrel = src * jnp.int32(NUM_REL) + rel
    chist = rel * jnp.int32(DPAD) + src

    c_t, deg_t = _sc_hist_stage(chist, dst)
    c5 = c_t.reshape(NW, NUM_REL, DPAD)
    wc = weight.transpose(1, 0, 2).reshape(DIM, NUM_REL * DIM)
    x_pad = jnp.pad(x, ((0, DPAD - N_NODES), (0, 0)))
    y, ab = _relu_xw_stats(x_pad, wc, c5, bn_gamma.reshape(1, DIM),
                           bn_beta.reshape(1, DIM))
    acc = _sc_acc_stage(srcrel, dst.reshape(NW, NCH, CH), y.reshape(KPAD, DIM))

    deg = deg_t.reshape(NW, DPAD).sum(axis=0)[:N_NODES].reshape(N_NODES, 1)
    return _c2_stage(acc, deg, ab)
